# baseline (speedup 1.0000x reference)
import functools

import jax
import jax.numpy as jnp
from jax import lax
from jax.experimental import pallas as pl
from jax.experimental.pallas import tpu as pltpu
from jax.experimental.pallas import tpu_sc as plsc

N, D = 262144, 256
H = D // 2
NUM_CORES = 2
NUM_SUBCORES = 16
NW = NUM_CORES * NUM_SUBCORES
ROWS_PER_W = N // NW
CHUNKS = 4
ROWS_PER_CHUNK = ROWS_PER_W // CHUNKS

_mesh = plsc.VectorSubcoreMesh(core_axis_name="c", subcore_axis_name="s")


@functools.partial(
    pl.kernel,
    mesh=_mesh,
    out_type=(
        jax.ShapeDtypeStruct((N, H), jnp.float32),
        jax.ShapeDtypeStruct((N, H), jnp.float32),
    ),
)
def _split_halves(inp_hbm, speed_hbm, dir_hbm):
    wid = lax.axis_index("s") * NUM_CORES + lax.axis_index("c")
    base = wid * ROWS_PER_W
    for i in range(CHUNKS):
        rows = pl.ds(base + i * ROWS_PER_CHUNK, ROWS_PER_CHUNK)
        pltpu.sync_copy(inp_hbm.at[rows, pl.ds(0, H)], speed_hbm.at[rows])
        pltpu.sync_copy(inp_hbm.at[rows, pl.ds(H, H)], dir_hbm.at[rows])


def kernel(inputs):
    return _split_halves(inputs)
